# baseline (speedup 1.0000x reference)
import functools

import jax
import jax.numpy as jnp
from jax import lax
from jax.experimental import pallas as pl
from jax.experimental.pallas import tpu as pltpu
from jax.experimental.pallas import tpu_sc as plsc

D = 768
H = 4 * D
T = 2048
V = 100000
VB = 512
RB = 512


def _layernorm(x, g, b):
    mu = jnp.mean(x, axis=-1, keepdims=True)
    var = jnp.mean((x - mu) ** 2, axis=-1, keepdims=True)
    return (x - mu) * lax.rsqrt(var + 1e-5) * g + b


def _bf(x):
    return x.astype(jnp.bfloat16)


def _gather_rows(wte, idx):
    info = plsc.get_sparse_core_info()
    nw = info.num_cores * info.num_subcores
    bpw = T // nw
    mesh = plsc.VectorSubcoreMesh(core_axis_name="c", subcore_axis_name="s")

    @functools.partial(
        pl.kernel,
        mesh=mesh,
        out_type=jax.ShapeDtypeStruct((T, D), jnp.float32),
        scratch_types=[
            pltpu.VMEM((bpw,), jnp.int32),
            pltpu.VMEM((bpw, D), jnp.float32),
            pltpu.SemaphoreType.DMA,
        ],
    )
    def k(table_hbm, idx_hbm, out_hbm, idx_v, rows_v, sem):
        wid = lax.axis_index("s") * info.num_cores + lax.axis_index("c")
        base = wid * bpw
        pltpu.sync_copy(idx_hbm.at[pl.ds(base, bpw)], idx_v)
        pltpu.async_copy(table_hbm.at[idx_v], rows_v, sem).wait()
        pltpu.sync_copy(rows_v, out_hbm.at[pl.ds(base, bpw)])

    return k(wte, idx)


def _prologue_body(x0_ref, rbg_ref, rbb_ref, fcw_ref, fcb_ref, pw_ref, pb_ref,
                   lfg_ref, lfb_ref, ow_ref, ob_ref, lg_ref, lb_ref, out_ref):
    xa = x0_ref[...]
    h = _layernorm(xa, rbg_ref[...], rbb_ref[...])
    h = jnp.dot(_bf(h), _bf(fcw_ref[...]), preferred_element_type=jnp.float32)
    h = h + fcb_ref[...]
    h = h * jax.nn.sigmoid(1.702 * h)
    h = jnp.dot(_bf(h), _bf(pw_ref[...]), preferred_element_type=jnp.float32)
    x = xa + h + pb_ref[...]
    x = _layernorm(x, lfg_ref[...], lfb_ref[...])
    for i in range(2):
        x = jnp.dot(_bf(x), _bf(ow_ref[i]), preferred_element_type=jnp.float32)
        x = x + ob_ref[i]
        x = _layernorm(x, lg_ref[i], lb_ref[i])
    out_ref[...] = _bf(x)


def _head_body(xf_ref, hw_ref, out_ref):
    out_ref[...] = jnp.dot(xf_ref[...], _bf(hw_ref[0]),
                           preferred_element_type=jnp.float32)


def kernel(idx, wte, rb_ln_g, rb_ln_b, rb_fc_w, rb_fc_b, rb_proj_w, rb_proj_b,
           ln_f_g, ln_f_b, lm_head_w, merge_ln_g, merge_ln_b, merge_fc_w,
           merge_fc_b, merge_proj_w, merge_proj_b, out_w, out_b, lnf_g, lnf_b,
           head_w):
    x0 = _gather_rows(wte, idx.astype(jnp.int32))

    row = lambda a: a.reshape(1, -1)
    xf = pl.pallas_call(
        _prologue_body,
        grid=(T // RB,),
        in_specs=[
            pl.BlockSpec((RB, D), lambda i: (i, 0)),
            pl.BlockSpec((1, D), lambda i: (0, 0)),
            pl.BlockSpec((1, D), lambda i: (0, 0)),
            pl.BlockSpec((D, H), lambda i: (0, 0)),
            pl.BlockSpec((1, H), lambda i: (0, 0)),
            pl.BlockSpec((H, D), lambda i: (0, 0)),
            pl.BlockSpec((1, D), lambda i: (0, 0)),
            pl.BlockSpec((1, D), lambda i: (0, 0)),
            pl.BlockSpec((1, D), lambda i: (0, 0)),
            pl.BlockSpec((2, D, D), lambda i: (0, 0, 0)),
            pl.BlockSpec((2, 1, D), lambda i: (0, 0, 0)),
            pl.BlockSpec((2, 1, D), lambda i: (0, 0, 0)),
            pl.BlockSpec((2, 1, D), lambda i: (0, 0, 0)),
        ],
        out_specs=pl.BlockSpec((RB, D), lambda i: (i, 0)),
        out_shape=jax.ShapeDtypeStruct((T, D), jnp.bfloat16),
        compiler_params=pltpu.CompilerParams(
            dimension_semantics=("arbitrary",),
        ),
    )(x0, row(rb_ln_g), row(rb_ln_b), rb_fc_w, row(rb_fc_b), rb_proj_w,
      row(rb_proj_b), row(ln_f_g), row(ln_f_b), out_w,
      out_b.reshape(2, 1, D), lnf_g.reshape(2, 1, D), lnf_b.reshape(2, 1, D))

    nv = pl.cdiv(V, VB)
    logits = pl.pallas_call(
        _head_body,
        grid=(nv,),
        in_specs=[
            pl.BlockSpec((T, D), lambda v: (0, 0)),
            pl.BlockSpec((1, D, VB), lambda v: (1, 0, v)),
        ],
        out_specs=pl.BlockSpec((T, VB), lambda v: (0, v)),
        out_shape=jax.ShapeDtypeStruct((T, V), jnp.float32),
        compiler_params=pltpu.CompilerParams(
            dimension_semantics=("arbitrary",),
        ),
    )(xf, head_w)
    return logits


# speedup vs baseline: 3.3197x; 3.3197x over previous
import functools

import jax
import jax.numpy as jnp
from jax import lax
from jax.experimental import pallas as pl
from jax.experimental.pallas import tpu as pltpu
from jax.experimental.pallas import tpu_sc as plsc

D = 768
H = 4 * D
T = 2048
V = 100000
VB = 512
RB = 512


def _layernorm(x, g, b):
    mu = jnp.mean(x, axis=-1, keepdims=True)
    var = jnp.mean((x - mu) ** 2, axis=-1, keepdims=True)
    return (x - mu) * lax.rsqrt(var + 1e-5) * g + b


def _bf(x):
    return x.astype(jnp.bfloat16)


def _gather_rows(wte, idx):
    info = plsc.get_sparse_core_info()
    nw = info.num_cores * info.num_subcores
    bpw = T // nw
    mesh = plsc.VectorSubcoreMesh(core_axis_name="c", subcore_axis_name="s")

    @functools.partial(
        pl.kernel,
        mesh=mesh,
        out_type=jax.ShapeDtypeStruct((T, D), jnp.float32),
        scratch_types=[
            pltpu.VMEM((bpw,), jnp.int32),
            pltpu.VMEM((bpw, D), jnp.float32),
            pltpu.SemaphoreType.DMA,
        ],
    )
    def k(table_hbm, idx_hbm, out_hbm, idx_v, rows_v, sem):
        wid = lax.axis_index("s") * info.num_cores + lax.axis_index("c")
        base = wid * bpw
        pltpu.sync_copy(idx_hbm.at[pl.ds(base, bpw)], idx_v)
        pltpu.async_copy(table_hbm.at[idx_v], rows_v, sem).wait()
        pltpu.sync_copy(rows_v, out_hbm.at[pl.ds(base, bpw)])

    return k(wte, idx)


def _prologue_body(x0_ref, rbg_ref, rbb_ref, fcw_ref, fcb_ref, pw_ref, pb_ref,
                   lfg_ref, lfb_ref, ow_ref, ob_ref, lg_ref, lb_ref, out_ref):
    xa = x0_ref[...]
    h = _layernorm(xa, rbg_ref[...], rbb_ref[...])
    h = jnp.dot(_bf(h), _bf(fcw_ref[...]), preferred_element_type=jnp.float32)
    h = h + fcb_ref[...]
    h = h * jax.nn.sigmoid(1.702 * h)
    h = jnp.dot(_bf(h), _bf(pw_ref[...]), preferred_element_type=jnp.float32)
    x = xa + h + pb_ref[...]
    x = _layernorm(x, lfg_ref[...], lfb_ref[...])
    for i in range(2):
        x = jnp.dot(_bf(x), _bf(ow_ref[i]), preferred_element_type=jnp.float32)
        x = x + ob_ref[i]
        x = _layernorm(x, lg_ref[i], lb_ref[i])
    out_ref[...] = _bf(x.T)


def _head_body(hw_ref, xft_ref, out_ref):
    out_ref[...] = jnp.dot(_bf(hw_ref[0]), xft_ref[...],
                           preferred_element_type=jnp.float32)


def kernel(idx, wte, rb_ln_g, rb_ln_b, rb_fc_w, rb_fc_b, rb_proj_w, rb_proj_b,
           ln_f_g, ln_f_b, lm_head_w, merge_ln_g, merge_ln_b, merge_fc_w,
           merge_fc_b, merge_proj_w, merge_proj_b, out_w, out_b, lnf_g, lnf_b,
           head_w):
    x0 = _gather_rows(wte, idx.astype(jnp.int32))

    row = lambda a: a.reshape(1, -1)
    xf = pl.pallas_call(
        _prologue_body,
        grid=(T // RB,),
        in_specs=[
            pl.BlockSpec((RB, D), lambda i: (i, 0)),
            pl.BlockSpec((1, D), lambda i: (0, 0)),
            pl.BlockSpec((1, D), lambda i: (0, 0)),
            pl.BlockSpec((D, H), lambda i: (0, 0)),
            pl.BlockSpec((1, H), lambda i: (0, 0)),
            pl.BlockSpec((H, D), lambda i: (0, 0)),
            pl.BlockSpec((1, D), lambda i: (0, 0)),
            pl.BlockSpec((1, D), lambda i: (0, 0)),
            pl.BlockSpec((1, D), lambda i: (0, 0)),
            pl.BlockSpec((2, D, D), lambda i: (0, 0, 0)),
            pl.BlockSpec((2, 1, D), lambda i: (0, 0, 0)),
            pl.BlockSpec((2, 1, D), lambda i: (0, 0, 0)),
            pl.BlockSpec((2, 1, D), lambda i: (0, 0, 0)),
        ],
        out_specs=pl.BlockSpec((D, RB), lambda i: (0, i)),
        out_shape=jax.ShapeDtypeStruct((D, T), jnp.bfloat16),
        compiler_params=pltpu.CompilerParams(
            dimension_semantics=("arbitrary",),
        ),
    )(x0, row(rb_ln_g), row(rb_ln_b), rb_fc_w, row(rb_fc_b), rb_proj_w,
      row(rb_proj_b), row(ln_f_g), row(ln_f_b), out_w,
      out_b.reshape(2, 1, D), lnf_g.reshape(2, 1, D), lnf_b.reshape(2, 1, D))

    hwt = jnp.transpose(head_w, (0, 2, 1))
    nv = pl.cdiv(V, VB)
    logits_t = pl.pallas_call(
        _head_body,
        grid=(nv,),
        in_specs=[
            pl.BlockSpec((1, VB, D), lambda v: (1, v, 0)),
            pl.BlockSpec((D, T), lambda v: (0, 0)),
        ],
        out_specs=pl.BlockSpec((VB, T), lambda v: (v, 0)),
        out_shape=jax.ShapeDtypeStruct((V, T), jnp.float32),
        compiler_params=pltpu.CompilerParams(
            dimension_semantics=("arbitrary",),
        ),
    )(hwt, xf)
    return logits_t.T


# speedup vs baseline: 3.8350x; 1.1552x over previous
import functools

import jax
import jax.numpy as jnp
from jax import lax
from jax.experimental import pallas as pl
from jax.experimental.pallas import tpu as pltpu
from jax.experimental.pallas import tpu_sc as plsc

D = 768
H = 4 * D
T = 2048
V = 100000
VB = 1024
RB = 512


def _layernorm(x, g, b):
    mu = jnp.mean(x, axis=-1, keepdims=True)
    var = jnp.mean((x - mu) ** 2, axis=-1, keepdims=True)
    return (x - mu) * lax.rsqrt(var + 1e-5) * g + b


def _bf(x):
    return x.astype(jnp.bfloat16)


def _gather_rows(wte, idx):
    info = plsc.get_sparse_core_info()
    nw = info.num_cores * info.num_subcores
    bpw = T // nw
    mesh = plsc.VectorSubcoreMesh(core_axis_name="c", subcore_axis_name="s")

    @functools.partial(
        pl.kernel,
        mesh=mesh,
        out_type=jax.ShapeDtypeStruct((T, D), jnp.float32),
        scratch_types=[
            pltpu.VMEM((bpw,), jnp.int32),
            pltpu.VMEM((bpw, D), jnp.float32),
            pltpu.SemaphoreType.DMA,
        ],
    )
    def k(table_hbm, idx_hbm, out_hbm, idx_v, rows_v, sem):
        wid = lax.axis_index("s") * info.num_cores + lax.axis_index("c")
        base = wid * bpw
        pltpu.sync_copy(idx_hbm.at[pl.ds(base, bpw)], idx_v)
        pltpu.async_copy(table_hbm.at[idx_v], rows_v, sem).wait()
        pltpu.sync_copy(rows_v, out_hbm.at[pl.ds(base, bpw)])

    return k(wte, idx)


def _prologue_body(x0_ref, rbg_ref, rbb_ref, fcw_ref, fcb_ref, pw_ref, pb_ref,
                   lfg_ref, lfb_ref, ow_ref, ob_ref, lg_ref, lb_ref, out_ref):
    xa = x0_ref[...]
    h = _layernorm(xa, rbg_ref[...], rbb_ref[...])
    h = jnp.dot(_bf(h), _bf(fcw_ref[...]), preferred_element_type=jnp.float32)
    h = h + fcb_ref[...]
    h = h * jax.nn.sigmoid(1.702 * h)
    h = jnp.dot(_bf(h), _bf(pw_ref[...]), preferred_element_type=jnp.float32)
    x = xa + h + pb_ref[...]
    x = _layernorm(x, lfg_ref[...], lfb_ref[...])
    for i in range(2):
        x = jnp.dot(_bf(x), _bf(ow_ref[i]), preferred_element_type=jnp.float32)
        x = x + ob_ref[i]
        x = _layernorm(x, lg_ref[i], lb_ref[i])
    out_ref[...] = _bf(x.T)


def _head_body(hw_ref, xft_ref, out_ref):
    out_ref[...] = jnp.dot(_bf(hw_ref[0]), xft_ref[...],
                           preferred_element_type=jnp.float32)


def kernel(idx, wte, rb_ln_g, rb_ln_b, rb_fc_w, rb_fc_b, rb_proj_w, rb_proj_b,
           ln_f_g, ln_f_b, lm_head_w, merge_ln_g, merge_ln_b, merge_fc_w,
           merge_fc_b, merge_proj_w, merge_proj_b, out_w, out_b, lnf_g, lnf_b,
           head_w):
    x0 = _gather_rows(wte, idx.astype(jnp.int32))

    row = lambda a: a.reshape(1, -1)
    xf = pl.pallas_call(
        _prologue_body,
        grid=(T // RB,),
        in_specs=[
            pl.BlockSpec((RB, D), lambda i: (i, 0)),
            pl.BlockSpec((1, D), lambda i: (0, 0)),
            pl.BlockSpec((1, D), lambda i: (0, 0)),
            pl.BlockSpec((D, H), lambda i: (0, 0)),
            pl.BlockSpec((1, H), lambda i: (0, 0)),
            pl.BlockSpec((H, D), lambda i: (0, 0)),
            pl.BlockSpec((1, D), lambda i: (0, 0)),
            pl.BlockSpec((1, D), lambda i: (0, 0)),
            pl.BlockSpec((1, D), lambda i: (0, 0)),
            pl.BlockSpec((2, D, D), lambda i: (0, 0, 0)),
            pl.BlockSpec((2, 1, D), lambda i: (0, 0, 0)),
            pl.BlockSpec((2, 1, D), lambda i: (0, 0, 0)),
            pl.BlockSpec((2, 1, D), lambda i: (0, 0, 0)),
        ],
        out_specs=pl.BlockSpec((D, RB), lambda i: (0, i)),
        out_shape=jax.ShapeDtypeStruct((D, T), jnp.bfloat16),
        compiler_params=pltpu.CompilerParams(
            dimension_semantics=("arbitrary",),
        ),
    )(x0, row(rb_ln_g), row(rb_ln_b), rb_fc_w, row(rb_fc_b), rb_proj_w,
      row(rb_proj_b), row(ln_f_g), row(ln_f_b), out_w,
      out_b.reshape(2, 1, D), lnf_g.reshape(2, 1, D), lnf_b.reshape(2, 1, D))

    hwt = jnp.transpose(head_w, (0, 2, 1))
    nv = pl.cdiv(V, VB)
    logits_t = pl.pallas_call(
        _head_body,
        grid=(nv,),
        in_specs=[
            pl.BlockSpec((1, VB, D), lambda v: (1, v, 0)),
            pl.BlockSpec((D, T), lambda v: (0, 0)),
        ],
        out_specs=pl.BlockSpec((VB, T), lambda v: (v, 0)),
        out_shape=jax.ShapeDtypeStruct((V, T), jnp.float32),
        compiler_params=pltpu.CompilerParams(
            dimension_semantics=("arbitrary",),
        ),
    )(hwt, xf)
    return logits_t.T


# speedup vs baseline: 4.1046x; 1.0703x over previous
import functools

import jax
import jax.numpy as jnp
from jax import lax
from jax.experimental import pallas as pl
from jax.experimental.pallas import tpu as pltpu
from jax.experimental.pallas import tpu_sc as plsc

D = 768
H = 4 * D
T = 2048
V = 100000
VB = 2048
RB = 512


def _layernorm(x, g, b):
    mu = jnp.mean(x, axis=-1, keepdims=True)
    var = jnp.mean((x - mu) ** 2, axis=-1, keepdims=True)
    return (x - mu) * lax.rsqrt(var + 1e-5) * g + b


def _bf(x):
    return x.astype(jnp.bfloat16)


def _gather_rows(wte, idx):
    info = plsc.get_sparse_core_info()
    nw = info.num_cores * info.num_subcores
    bpw = T // nw
    mesh = plsc.VectorSubcoreMesh(core_axis_name="c", subcore_axis_name="s")

    @functools.partial(
        pl.kernel,
        mesh=mesh,
        out_type=jax.ShapeDtypeStruct((T, D), jnp.float32),
        scratch_types=[
            pltpu.VMEM((bpw,), jnp.int32),
            pltpu.VMEM((bpw, D), jnp.float32),
            pltpu.SemaphoreType.DMA,
        ],
    )
    def k(table_hbm, idx_hbm, out_hbm, idx_v, rows_v, sem):
        wid = lax.axis_index("s") * info.num_cores + lax.axis_index("c")
        base = wid * bpw
        pltpu.sync_copy(idx_hbm.at[pl.ds(base, bpw)], idx_v)
        pltpu.async_copy(table_hbm.at[idx_v], rows_v, sem).wait()
        pltpu.sync_copy(rows_v, out_hbm.at[pl.ds(base, bpw)])

    return k(wte, idx)


def _prologue_body(x0_ref, rbg_ref, rbb_ref, fcw_ref, fcb_ref, pw_ref, pb_ref,
                   lfg_ref, lfb_ref, ow_ref, ob_ref, lg_ref, lb_ref, out_ref):
    xa = x0_ref[...]
    h = _layernorm(xa, rbg_ref[...], rbb_ref[...])
    h = jnp.dot(_bf(h), _bf(fcw_ref[...]), preferred_element_type=jnp.float32)
    h = h + fcb_ref[...]
    h = h * jax.nn.sigmoid(1.702 * h)
    h = jnp.dot(_bf(h), _bf(pw_ref[...]), preferred_element_type=jnp.float32)
    x = xa + h + pb_ref[...]
    x = _layernorm(x, lfg_ref[...], lfb_ref[...])
    for i in range(2):
        x = jnp.dot(_bf(x), _bf(ow_ref[i]), preferred_element_type=jnp.float32)
        x = x + ob_ref[i]
        x = _layernorm(x, lg_ref[i], lb_ref[i])
    out_ref[...] = _bf(x.T)


def _head_body(hw_ref, xft_ref, out_ref):
    out_ref[...] = jnp.dot(_bf(hw_ref[0]), xft_ref[...],
                           preferred_element_type=jnp.float32)


def kernel(idx, wte, rb_ln_g, rb_ln_b, rb_fc_w, rb_fc_b, rb_proj_w, rb_proj_b,
           ln_f_g, ln_f_b, lm_head_w, merge_ln_g, merge_ln_b, merge_fc_w,
           merge_fc_b, merge_proj_w, merge_proj_b, out_w, out_b, lnf_g, lnf_b,
           head_w):
    x0 = _gather_rows(wte, idx.astype(jnp.int32))

    row = lambda a: a.reshape(1, -1)
    xf = pl.pallas_call(
        _prologue_body,
        grid=(T // RB,),
        in_specs=[
            pl.BlockSpec((RB, D), lambda i: (i, 0)),
            pl.BlockSpec((1, D), lambda i: (0, 0)),
            pl.BlockSpec((1, D), lambda i: (0, 0)),
            pl.BlockSpec((D, H), lambda i: (0, 0)),
            pl.BlockSpec((1, H), lambda i: (0, 0)),
            pl.BlockSpec((H, D), lambda i: (0, 0)),
            pl.BlockSpec((1, D), lambda i: (0, 0)),
            pl.BlockSpec((1, D), lambda i: (0, 0)),
            pl.BlockSpec((1, D), lambda i: (0, 0)),
            pl.BlockSpec((2, D, D), lambda i: (0, 0, 0)),
            pl.BlockSpec((2, 1, D), lambda i: (0, 0, 0)),
            pl.BlockSpec((2, 1, D), lambda i: (0, 0, 0)),
            pl.BlockSpec((2, 1, D), lambda i: (0, 0, 0)),
        ],
        out_specs=pl.BlockSpec((D, RB), lambda i: (0, i)),
        out_shape=jax.ShapeDtypeStruct((D, T), jnp.bfloat16),
        compiler_params=pltpu.CompilerParams(
            dimension_semantics=("arbitrary",),
        ),
    )(x0, row(rb_ln_g), row(rb_ln_b), rb_fc_w, row(rb_fc_b), rb_proj_w,
      row(rb_proj_b), row(ln_f_g), row(ln_f_b), out_w,
      out_b.reshape(2, 1, D), lnf_g.reshape(2, 1, D), lnf_b.reshape(2, 1, D))

    hwt = jnp.transpose(head_w, (0, 2, 1))
    nv = pl.cdiv(V, VB)
    logits_t = pl.pallas_call(
        _head_body,
        grid=(nv,),
        in_specs=[
            pl.BlockSpec((1, VB, D), lambda v: (1, v, 0)),
            pl.BlockSpec((D, T), lambda v: (0, 0)),
        ],
        out_specs=pl.BlockSpec((VB, T), lambda v: (v, 0)),
        out_shape=jax.ShapeDtypeStruct((V, T), jnp.float32),
        compiler_params=pltpu.CompilerParams(
            dimension_semantics=("arbitrary",),
        ),
    )(hwt, xf)
    return logits_t.T
